# baseline (speedup 1.0000x reference)
import functools

import jax
import jax.numpy as jnp
from jax import lax
from jax.experimental import pallas as pl
from jax.experimental.pallas import tpu as pltpu
from jax.experimental.pallas import tpu_sc as plsc

BATCH = 16384
EMBED_DIM = 16
LANES = 16
NUM_CORES = 2
NUM_SUBCORES = 16
NW = NUM_CORES * NUM_SUBCORES
BPW = BATCH // NW
GROUPS = BPW // LANES
VOCAB = 1000000


def _sc_partial_sums(input_user, input_item, pred_data_label,
                     user_flat, item_flat, bias_tab):
  mesh = plsc.VectorSubcoreMesh(core_axis_name="c", subcore_axis_name="s")

  @functools.partial(
      pl.kernel,
      out_type=[
          jax.ShapeDtypeStruct((NW * LANES,), jnp.float32),
          jax.ShapeDtypeStruct((NW * LANES,), jnp.float32),
      ],
      mesh=mesh,
      compiler_params=pltpu.CompilerParams(needs_layout_passes=False),
      scratch_types=[
          pltpu.VMEM((BPW,), jnp.int32),
          pltpu.VMEM((BPW,), jnp.int32),
          pltpu.VMEM((BPW,), jnp.int32),
          pltpu.VMEM((EMBED_DIM * BPW,), jnp.int32),
          pltpu.VMEM((EMBED_DIM * BPW,), jnp.int32),
          pltpu.VMEM((EMBED_DIM * BPW,), jnp.int32),
          pltpu.VMEM((EMBED_DIM * BPW,), jnp.float32),
          pltpu.VMEM((EMBED_DIM * BPW,), jnp.float32),
          pltpu.VMEM((EMBED_DIM * BPW,), jnp.float32),
          pltpu.VMEM((BPW,), jnp.float32),
          pltpu.VMEM((BPW,), jnp.float32),
          pltpu.VMEM((LANES,), jnp.float32),
          pltpu.SemaphoreType.DMA,
          pltpu.SemaphoreType.DMA,
          pltpu.SemaphoreType.DMA,
          pltpu.SemaphoreType.DMA,
          pltpu.SemaphoreType.DMA,
      ],
  )
  def sc_kernel(uidx_hbm, iidx_hbm, gidx_hbm, uflat_hbm, iflat_hbm, btab_hbm,
                out_s1, out_s2,
                idx_u, idx_i, idx_g, off_u, off_i, off_g,
                u_vals, i_vals, g_vals, bias_i, bias_g, acc_st,
                sem_u, sem_i, sem_g, sem_bi, sem_bg):
    wid = lax.axis_index("s") * NUM_CORES + lax.axis_index("c")
    base = wid * BPW

    pltpu.sync_copy(uidx_hbm.at[pl.ds(base, BPW)], idx_u)
    pltpu.sync_copy(iidx_hbm.at[pl.ds(base, BPW)], idx_i)
    pltpu.sync_copy(gidx_hbm.at[pl.ds(base, BPW)], idx_g)

    def fill_offsets(idx_ref, off_ref):
      def body(t, _):
        sl = pl.ds(t * LANES, LANES)
        iv = idx_ref[sl]
        for d in range(EMBED_DIM):
          off_ref[pl.ds(d * BPW + t * LANES, LANES)] = iv + d * VOCAB
        return 0

      lax.fori_loop(0, GROUPS, body, 0)

    fill_offsets(idx_u, off_u)
    fill_offsets(idx_i, off_i)
    fill_offsets(idx_g, off_g)

    copies = [
        pltpu.async_copy(btab_hbm.at[idx_i], bias_i, sem_bi),
        pltpu.async_copy(btab_hbm.at[idx_g], bias_g, sem_bg),
    ]
    for d in range(EMBED_DIM):
      dsl = pl.ds(d * BPW, BPW)
      copies.append(pltpu.async_copy(
          uflat_hbm.at[off_u.at[dsl]], u_vals.at[dsl], sem_u))
      copies.append(pltpu.async_copy(
          iflat_hbm.at[off_i.at[dsl]], i_vals.at[dsl], sem_i))
      copies.append(pltpu.async_copy(
          iflat_hbm.at[off_g.at[dsl]], g_vals.at[dsl], sem_g))
    for c in copies:
      c.wait()

    zero = jnp.zeros((LANES,), jnp.float32)

    def dot_body(g, carry):
      a1, a2 = carry
      for d in range(EMBED_DIM):
        sl = pl.ds(d * BPW + g * LANES, LANES)
        u = u_vals[sl]
        a1 = a1 + u * i_vals[sl]
        a2 = a2 + u * g_vals[sl]
      return a1, a2

    acc1, acc2 = lax.fori_loop(0, GROUPS, dot_body, (zero, zero))

    def bias_body(t, carry):
      b1, b2 = carry
      sl = pl.ds(t * LANES, LANES)
      return b1 + bias_i[sl], b2 + bias_g[sl]

    b1, b2 = lax.fori_loop(0, GROUPS, bias_body, (zero, zero))

    acc_st[...] = acc1 + b1
    pltpu.sync_copy(acc_st, out_s1.at[pl.ds(wid * LANES, LANES)])
    acc_st[...] = acc2 + b2
    pltpu.sync_copy(acc_st, out_s2.at[pl.ds(wid * LANES, LANES)])

  return sc_kernel(input_user, input_item, pred_data_label,
                   user_flat, item_flat, bias_tab)


def _tc_loss(s1_partials, s2_partials):
  def body(s1_ref, s2_ref, out_ref):
    inv_b = 1.0 / float(BATCH)
    s1 = jnp.sum(s1_ref[...]) * inv_b
    s2 = jnp.sum(s2_ref[...]) * inv_b

    def softplus(x):
      return jnp.maximum(x, 0.0) + jnp.log(1.0 + jnp.exp(-jnp.abs(x)))

    out_ref[...] = jnp.full((1, 1), softplus(-s2) + softplus(s1))

  out = pl.pallas_call(
      body,
      out_shape=jax.ShapeDtypeStruct((1, 1), jnp.float32),
  )(s1_partials, s2_partials)
  return out[0, 0]


def kernel(input_user, input_item, pred_data_label,
           D_user_embeddings, D_item_embeddings, D_item_bias):
  user_flat = D_user_embeddings.T.reshape(VOCAB * EMBED_DIM)
  item_flat = D_item_embeddings.T.reshape(VOCAB * EMBED_DIM)
  s1, s2 = _sc_partial_sums(input_user, input_item, pred_data_label,
                            user_flat, item_flat, D_item_bias)
  return _tc_loss(s1.reshape(4, 128), s2.reshape(4, 128))


# speedup vs baseline: 13.4621x; 13.4621x over previous
import functools

import jax
import jax.numpy as jnp
from jax import lax
from jax.experimental import pallas as pl
from jax.experimental.pallas import tpu as pltpu
from jax.experimental.pallas import tpu_sc as plsc

BATCH = 16384
EMBED_DIM = 16
LANES = 16
NUM_CORES = 2
NUM_SUBCORES = 16
NW = NUM_CORES * NUM_SUBCORES
BPW = BATCH // NW
GROUPS = BPW // LANES
VOCAB = 1000000
GB = 16
NG = BPW // GB


def _sc_partial_sums(input_user, input_item, pred_data_label,
                     user_tab_t, item_tab_t, bias_tab):
  mesh = plsc.VectorSubcoreMesh(core_axis_name="c", subcore_axis_name="s")

  @functools.partial(
      pl.kernel,
      out_type=[
          jax.ShapeDtypeStruct((NW * LANES,), jnp.float32),
          jax.ShapeDtypeStruct((NW * LANES,), jnp.float32),
      ],
      mesh=mesh,
      compiler_params=pltpu.CompilerParams(needs_layout_passes=False),
      scratch_types=[
          pltpu.VMEM((BPW,), jnp.int32),
          pltpu.VMEM((BPW,), jnp.int32),
          pltpu.VMEM((BPW,), jnp.int32),
          [pltpu.VMEM((2, 8, 128), jnp.float32) for _ in range(GB)],
          [pltpu.VMEM((2, 8, 128), jnp.float32) for _ in range(GB)],
          pltpu.VMEM((EMBED_DIM * BPW,), jnp.float32),
          pltpu.VMEM((BPW,), jnp.float32),
          pltpu.VMEM((BPW,), jnp.float32),
          pltpu.VMEM((LANES,), jnp.float32),
          pltpu.SemaphoreType.DMA,
          pltpu.SemaphoreType.DMA,
          pltpu.SemaphoreType.DMA,
          pltpu.SemaphoreType.DMA,
      ],
  )
  def sc_kernel(uidx_hbm, iidx_hbm, gidx_hbm, utab_hbm, itab_hbm, btab_hbm,
                out_s1, out_s2,
                idx_vm, idx_i, idx_g, bank_a, bank_b,
                u_vals, bias_i, bias_g, acc_st,
                sem_a, sem_b, sem_bi, sem_bg):
    wid = lax.axis_index("s") * NUM_CORES + lax.axis_index("c")
    base = wid * BPW

    pltpu.sync_copy(iidx_hbm.at[pl.ds(base, BPW)], idx_i)
    pltpu.sync_copy(gidx_hbm.at[pl.ds(base, BPW)], idx_g)

    cbi = pltpu.async_copy(btab_hbm.at[idx_i], bias_i, sem_bi)
    cbg = pltpu.async_copy(btab_hbm.at[idx_g], bias_g, sem_bg)

    utab3 = utab_hbm.reshape(2, 8, VOCAB)
    itab3 = itab_hbm.reshape(2, 8, VOCAB)

    iota16 = lax.iota(jnp.int32, LANES)
    rv = lax.shift_right_logical(iota16, 3)
    sv = jnp.bitwise_and(iota16, 7)
    zero = jnp.zeros((LANES,), jnp.float32)

    def fire(g, bank, sem, tab3):
      @pl.when(g < NG)
      def _():
        iv = idx_vm[pl.ds(g * GB, GB)]
        for b in range(GB):
          off = pl.multiple_of(
              lax.shift_left(lax.shift_right_logical(iv[b], 7), 7), 128)
          pltpu.async_copy(tab3.at[:, :, pl.ds(off, 128)], bank[b], sem)

    def proc(g, bank, sem, tab3, acc, store):
      iv = idx_vm[pl.ds(g * GB, GB)]
      lanes = jnp.bitwise_and(iv, 127)
      for b in range(GB):
        pltpu.make_async_copy(tab3.at[:, :, pl.ds(0, 128)],
                              bank[b], sem).wait()
        j = g * GB + b
        lane = jnp.full((LANES,), lanes[b], jnp.int32)
        v = plsc.load_gather(bank[b], [rv, sv, lane])
        if store:
          u_vals[pl.ds(j * EMBED_DIM, EMBED_DIM)] = v
        else:
          acc = acc + v * u_vals[pl.ds(j * EMBED_DIM, EMBED_DIM)]
      return acc

    def phase(idx_hbm, tab3, store):
      pltpu.sync_copy(idx_hbm.at[pl.ds(base, BPW)], idx_vm)
      fire(0, bank_a, sem_a, tab3)

      def body(j2, acc):
        g0 = 2 * j2
        fire(g0 + 1, bank_b, sem_b, tab3)
        acc = proc(g0, bank_a, sem_a, tab3, acc, store)
        fire(g0 + 2, bank_a, sem_a, tab3)
        acc = proc(g0 + 1, bank_b, sem_b, tab3, acc, store)
        return acc

      return lax.fori_loop(0, NG // 2, body, zero)

    phase(uidx_hbm, utab3, True)
    acc1 = phase(iidx_hbm, itab3, False)
    acc2 = phase(gidx_hbm, itab3, False)

    cbi.wait()
    cbg.wait()

    def bias_body(t, carry):
      b1, b2 = carry
      sl = pl.ds(t * LANES, LANES)
      return b1 + bias_i[sl], b2 + bias_g[sl]

    b1, b2 = lax.fori_loop(0, GROUPS, bias_body, (zero, zero))

    acc_st[...] = acc1 + b1
    pltpu.sync_copy(acc_st, out_s1.at[pl.ds(wid * LANES, LANES)])
    acc_st[...] = acc2 + b2
    pltpu.sync_copy(acc_st, out_s2.at[pl.ds(wid * LANES, LANES)])

  return sc_kernel(input_user, input_item, pred_data_label,
                   user_tab_t, item_tab_t, bias_tab)


def _tc_loss(s1_partials, s2_partials):
  def body(s1_ref, s2_ref, out_ref):
    inv_b = 1.0 / float(BATCH)
    s1 = jnp.sum(s1_ref[...]) * inv_b
    s2 = jnp.sum(s2_ref[...]) * inv_b

    def softplus(x):
      return jnp.maximum(x, 0.0) + jnp.log(1.0 + jnp.exp(-jnp.abs(x)))

    out_ref[...] = jnp.full((1, 1), softplus(-s2) + softplus(s1))

  out = pl.pallas_call(
      body,
      out_shape=jax.ShapeDtypeStruct((1, 1), jnp.float32),
  )(s1_partials, s2_partials)
  return out[0, 0]


def kernel(input_user, input_item, pred_data_label,
           D_user_embeddings, D_item_embeddings, D_item_bias):
  s1, s2 = _sc_partial_sums(input_user, input_item, pred_data_label,
                            D_user_embeddings.T, D_item_embeddings.T,
                            D_item_bias)
  return _tc_loss(s1.reshape(4, 128), s2.reshape(4, 128))
